# Initial kernel scaffold; baseline (speedup 1.0000x reference)
import functools
import math

import jax
import jax.numpy as jnp
import numpy as np
from jax import lax
from jax.experimental import pallas as pl
from jax.experimental.pallas import tpu as pltpu
from jax.experimental.pallas import tpu_sc as plsc

_VOCAB = 100000
_D = 64
_B = 4096
_L = 200
_SPE1 = 3
_SPE2 = 4
_N = _B * _L
_NC = 2
_NS = 16
_NW = _NC * _NS
_PER_W = _N // _NW
_C = 512
_G = _PER_W // _C
_SCALE = math.sqrt(_D)


def _positional_encoding_np():
    pos = np.arange(_L, dtype=np.float32)[:, None]
    div = np.exp(np.arange(0, _D, 2, dtype=np.float32) * (-math.log(10000.0) / _D))
    pe = np.zeros((_L, _D), dtype=np.float32)
    pe[:, 0::2] = np.sin(pos * div)
    pe[:, 1::2] = np.cos(pos * div)
    return pe


_PE = _positional_encoding_np()

_mesh = plsc.VectorSubcoreMesh(core_axis_name="c", subcore_axis_name="s")


@functools.partial(
    pl.kernel,
    out_type=jax.ShapeDtypeStruct((_N, _D), jnp.float32),
    mesh=_mesh,
    scratch_types=[
        pltpu.VMEM((3 * _L, _D), jnp.float32),
        pltpu.VMEM((_C, _D), jnp.float32),
        pltpu.VMEM((_C,), jnp.int32),
        pltpu.VMEM((_C,), jnp.int32),
        pltpu.VMEM((70, _D), jnp.float32),
        pltpu.VMEM((2, _D), jnp.float32),
        pltpu.SemaphoreType.DMA,
    ],
)
def _sc_kernel(ctx_hbm, seg_hbm, pnt_hbm, tag_hbm, table_hbm, pe_hbm, out_hbm,
               comb, dest, idxb, segb, prow, ps, sem):
    wid = lax.axis_index("s") * _NC + lax.axis_index("c")

    pltpu.sync_copy(pe_hbm, comb.at[pl.ds(0, _L)])
    for p in range(2):
        pltpu.sync_copy(pnt_hbm.at[p], idxb.at[pl.ds(0, 50)])
        pltpu.sync_copy(tag_hbm.at[p], idxb.at[pl.ds(56, 20)])
        pltpu.async_copy(table_hbm.at[idxb.at[pl.ds(0, 50)]],
                         prow.at[pl.ds(0, 50)], sem).wait()
        pltpu.async_copy(table_hbm.at[idxb.at[pl.ds(56, 20)]],
                         prow.at[pl.ds(50, 20)], sem).wait()
        for j in range(4):
            def _acc_body(i, a, j=j):
                return a + prow[i, pl.ds(j * 16, 16)]
            acc = lax.fori_loop(0, 70, _acc_body, jnp.zeros((16,), jnp.float32))
            ps[p, pl.ds(j * 16, 16)] = acc

    def _comb_body(l, _):
        for j in range(4):
            v = comb[l, pl.ds(j * 16, 16)]
            comb[l + _L, pl.ds(j * 16, 16)] = v + ps[0, pl.ds(j * 16, 16)]
            comb[l + 2 * _L, pl.ds(j * 16, 16)] = v + ps[1, pl.ds(j * 16, 16)]
        return 0
    lax.fori_loop(0, _L, _comb_body, 0)

    base_w = wid * _PER_W

    def _chunk_body(g, _):
        base = base_w + g * _C
        pltpu.sync_copy(ctx_hbm.at[pl.ds(base, _C)], idxb)
        pltpu.sync_copy(seg_hbm.at[pl.ds(base, _C)], segb)
        cps = [pltpu.async_copy(table_hbm.at[idxb.at[pl.ds(j * 128, 128)]],
                                dest.at[pl.ds(j * 128, 128)], sem)
               for j in range(_C // 128)]
        for cp in cps:
            cp.wait()

        def _row_body(r, _):
            s = segb[r]
            l = lax.rem(base + r, _L)
            k = l + jnp.where(s == _SPE1, _L,
                              jnp.where(s == _SPE2, 2 * _L, 0))
            for j in range(4):
                d = dest[r, pl.ds(j * 16, 16)]
                c = comb[k, pl.ds(j * 16, 16)]
                dest[r, pl.ds(j * 16, 16)] = d * _SCALE + c
            return 0
        lax.fori_loop(0, _C, _row_body, 0)
        pltpu.sync_copy(dest, out_hbm.at[pl.ds(base, _C)])
        return 0
    lax.fori_loop(0, _G, _chunk_body, 0)


def kernel(context, segs, personas_no_tag, tags, table):
    pe = jnp.asarray(_PE)
    out = _sc_kernel(
        context.reshape(-1).astype(jnp.int32),
        segs.reshape(-1).astype(jnp.int32),
        personas_no_tag.astype(jnp.int32),
        tags.astype(jnp.int32),
        table.astype(jnp.float32),
        pe,
    )
    return out.reshape(_B, _L, _D)


# speedup vs baseline: 2.3491x; 2.3491x over previous
import functools
import math

import jax
import jax.numpy as jnp
import numpy as np
from jax import lax
from jax.experimental import pallas as pl
from jax.experimental.pallas import tpu as pltpu
from jax.experimental.pallas import tpu_sc as plsc

_VOCAB = 100000
_D = 64
_B = 4096
_L = 200
_SPE1 = 3
_SPE2 = 4
_N = _B * _L
_NC = 2
_NS = 16
_NW = _NC * _NS
_PER_W = _N // _NW
_C = 512
_G = _PER_W // _C
_SCALE = math.sqrt(_D)


def _positional_encoding_np():
    pos = np.arange(_L, dtype=np.float32)[:, None]
    div = np.exp(np.arange(0, _D, 2, dtype=np.float32) * (-math.log(10000.0) / _D))
    pe = np.zeros((_L, _D), dtype=np.float32)
    pe[:, 0::2] = np.sin(pos * div)
    pe[:, 1::2] = np.cos(pos * div)
    return pe


_PE = _positional_encoding_np()

_mesh = plsc.VectorSubcoreMesh(core_axis_name="c", subcore_axis_name="s")


@functools.partial(
    pl.kernel,
    out_type=jax.ShapeDtypeStruct((_N, _D), jnp.float32),
    mesh=_mesh,
    compiler_params=pltpu.CompilerParams(use_tc_tiling_on_sc=False),
    scratch_types=[
        pltpu.VMEM((3 * _L, _D), jnp.float32),
        pltpu.VMEM((_C, _D), jnp.float32),
        pltpu.VMEM((_C,), jnp.int32),
        pltpu.VMEM((_C,), jnp.int32),
        pltpu.VMEM((72, _D), jnp.float32),
        pltpu.VMEM((2, _D), jnp.float32),
        pltpu.SemaphoreType.DMA,
    ],
)
def _sc_kernel(ctx_hbm, seg_hbm, pidx_hbm, table_hbm, pe_hbm, out_hbm,
               comb, dest, idxb, segb, prow, ps, sem):
    wid = lax.axis_index("s") * _NC + lax.axis_index("c")

    pltpu.sync_copy(pe_hbm, comb.at[pl.ds(0, _L)])
    for p in range(2):
        pltpu.sync_copy(pidx_hbm.at[pl.ds(p * 72, 72)], idxb.at[pl.ds(0, 72)])
        pltpu.async_copy(table_hbm.at[idxb.at[pl.ds(0, 72)]],
                         prow.at[pl.ds(0, 72)], sem).wait()
        for j in range(4):
            def _acc_body(i, a, j=j):
                return a + prow[i, pl.ds(j * 16, 16)]
            acc = lax.fori_loop(0, 70, _acc_body, jnp.zeros((16,), jnp.float32))
            ps[p, pl.ds(j * 16, 16)] = acc

    def _comb_body(l, _):
        for j in range(4):
            v = comb[l, pl.ds(j * 16, 16)]
            comb[l + _L, pl.ds(j * 16, 16)] = v + ps[0, pl.ds(j * 16, 16)]
            comb[l + 2 * _L, pl.ds(j * 16, 16)] = v + ps[1, pl.ds(j * 16, 16)]
        return 0
    lax.fori_loop(0, _L, _comb_body, 0)

    base_w = wid * _PER_W

    def _chunk_body(g, _):
        base = base_w + g * _C
        pltpu.sync_copy(ctx_hbm.at[pl.ds(base, _C)], idxb)
        pltpu.sync_copy(seg_hbm.at[pl.ds(base, _C)], segb)
        cps = [pltpu.async_copy(table_hbm.at[idxb.at[pl.ds(j * 128, 128)]],
                                dest.at[pl.ds(j * 128, 128)], sem)
               for j in range(_C // 128)]
        for cp in cps:
            cp.wait()

        def _grp_body(t, _):
            r0 = t * 16
            sv = segb[pl.ds(r0, 16)]
            lv = lax.rem(base + r0 + lax.iota(jnp.int32, 16), _L)
            kv = lv + jnp.where(sv == _SPE1, _L,
                                jnp.where(sv == _SPE2, 2 * _L, 0))
            for i in range(16):
                k = kv[i]
                for j in range(4):
                    d = dest[r0 + i, pl.ds(j * 16, 16)]
                    c = comb[k, pl.ds(j * 16, 16)]
                    dest[r0 + i, pl.ds(j * 16, 16)] = d * _SCALE + c
            return 0
        lax.fori_loop(0, _C // 16, _grp_body, 0)
        pltpu.sync_copy(dest, out_hbm.at[pl.ds(base, _C)])
        return 0
    lax.fori_loop(0, _G, _chunk_body, 0)


def kernel(context, segs, personas_no_tag, tags, table):
    pe = jnp.asarray(_PE)
    pnt = personas_no_tag.astype(jnp.int32)
    tg = tags.astype(jnp.int32)
    z2 = jnp.zeros((2,), jnp.int32)
    pidx = jnp.concatenate([pnt[0], tg[0], z2, pnt[1], tg[1], z2])
    out = _sc_kernel(
        context.reshape(-1).astype(jnp.int32),
        segs.reshape(-1).astype(jnp.int32),
        pidx,
        table.astype(jnp.float32),
        pe,
    )
    return out.reshape(_B, _L, _D)


# speedup vs baseline: 2.6594x; 1.1321x over previous
import functools
import math

import jax
import jax.numpy as jnp
import numpy as np
from jax import lax
from jax.experimental import pallas as pl
from jax.experimental.pallas import tpu as pltpu
from jax.experimental.pallas import tpu_sc as plsc

_VOCAB = 100000
_D = 64
_B = 4096
_L = 200
_SPE1 = 3
_SPE2 = 4
_N = _B * _L
_NC = 2
_NS = 16
_NW = _NC * _NS
_PER_W = _N // _NW
_C = 512
_G = _PER_W // _C
_SCALE = math.sqrt(_D)


def _positional_encoding_np():
    pos = np.arange(_L, dtype=np.float32)[:, None]
    div = np.exp(np.arange(0, _D, 2, dtype=np.float32) * (-math.log(10000.0) / _D))
    pe = np.zeros((_L, _D), dtype=np.float32)
    pe[:, 0::2] = np.sin(pos * div)
    pe[:, 1::2] = np.cos(pos * div)
    return pe


_PE = _positional_encoding_np()

_mesh = plsc.VectorSubcoreMesh(core_axis_name="c", subcore_axis_name="s")


@functools.partial(
    pl.kernel,
    out_type=jax.ShapeDtypeStruct((_N, _D), jnp.float32),
    mesh=_mesh,
    compiler_params=pltpu.CompilerParams(use_tc_tiling_on_sc=False),
    scratch_types=[
        pltpu.VMEM((3 * _L, _D), jnp.float32),
        pltpu.VMEM((2, _C, _D), jnp.float32),
        pltpu.VMEM((2, _C), jnp.int32),
        pltpu.VMEM((2, _C), jnp.int32),
        pltpu.VMEM((72, _D), jnp.float32),
        pltpu.VMEM((2, _D), jnp.float32),
        pltpu.SemaphoreType.DMA,
        pltpu.SemaphoreType.DMA,
        pltpu.SemaphoreType.DMA,
    ],
)
def _sc_kernel(ctx_hbm, seg_hbm, pidx_hbm, table_hbm, pe_hbm, out_hbm,
               comb, dest, idxb, segb, prow, ps, isem, gsem, osem):
    wid = lax.axis_index("s") * _NC + lax.axis_index("c")

    pltpu.sync_copy(pe_hbm, comb.at[pl.ds(0, _L)])
    for p in range(2):
        pltpu.sync_copy(pidx_hbm.at[pl.ds(p * 72, 72)],
                        idxb.at[0, pl.ds(0, 72)])
        pltpu.async_copy(table_hbm.at[idxb.at[0, pl.ds(0, 72)]],
                         prow.at[pl.ds(0, 72)], gsem).wait()
        for j in range(4):
            def _acc_body(i, a, j=j):
                return a + prow[i, pl.ds(j * 16, 16)]
            acc = lax.fori_loop(0, 70, _acc_body, jnp.zeros((16,), jnp.float32))
            ps[p, pl.ds(j * 16, 16)] = acc

    def _comb_body(l, _):
        for j in range(4):
            v = comb[l, pl.ds(j * 16, 16)]
            comb[l + _L, pl.ds(j * 16, 16)] = v + ps[0, pl.ds(j * 16, 16)]
            comb[l + 2 * _L, pl.ds(j * 16, 16)] = v + ps[1, pl.ds(j * 16, 16)]
        return 0
    lax.fori_loop(0, _L, _comb_body, 0)

    base_w = wid * _PER_W

    def _fire_idx(q, slot):
        base = base_w + q * _C
        pltpu.async_copy(ctx_hbm.at[pl.ds(base, _C)], idxb.at[slot], isem)
        pltpu.async_copy(seg_hbm.at[pl.ds(base, _C)], segb.at[slot], isem)

    def _wait_idx(slot):
        pltpu.make_async_copy(ctx_hbm.at[pl.ds(0, _C)], idxb.at[slot],
                              isem).wait()
        pltpu.make_async_copy(seg_hbm.at[pl.ds(0, _C)], segb.at[slot],
                              isem).wait()

    def _fire_gather(q, slot):
        for j in range(_C // 128):
            pltpu.async_copy(
                table_hbm.at[idxb.at[slot, pl.ds(j * 128, 128)]],
                dest.at[slot, pl.ds(j * 128, 128)], gsem)

    def _wait_gather(slot):
        pltpu.make_async_copy(out_hbm.at[pl.ds(0, _C)], dest.at[slot],
                              gsem).wait()

    def _fire_out(q, slot):
        base = base_w + q * _C
        pltpu.async_copy(dest.at[slot], out_hbm.at[pl.ds(base, _C)], osem)

    def _wait_out():
        pltpu.make_async_copy(dest.at[0], out_hbm.at[pl.ds(0, _C)],
                              osem).wait()

    def _compute(q, slot):
        base = base_w + q * _C
        db = dest.at[slot]
        sb = segb.at[slot]

        def _grp_body(t, _):
            r0 = t * 16
            sv = sb[pl.ds(r0, 16)]
            lv = lax.rem(base + r0 + lax.iota(jnp.int32, 16), _L)
            kv = lv + jnp.where(sv == _SPE1, _L,
                                jnp.where(sv == _SPE2, 2 * _L, 0))
            for i in range(16):
                k = kv[i]
                for j in range(4):
                    d = db[r0 + i, pl.ds(j * 16, 16)]
                    c = comb[k, pl.ds(j * 16, 16)]
                    db[r0 + i, pl.ds(j * 16, 16)] = d * _SCALE + c
            return 0
        lax.fori_loop(0, _C // 16, _grp_body, 0)

    def _iter(q, b):
        nb = 1 - b

        @pl.when(q < _G - 1)
        def _():
            _wait_idx(nb)

        @pl.when(q >= 1)
        def _():
            _wait_out()

        @pl.when(q < _G - 1)
        def _():
            _fire_gather(q + 1, nb)

        _wait_gather(b)
        _compute(q, b)

        @pl.when(q < _G - 2)
        def _():
            _fire_idx(q + 2, b)

        _fire_out(q, b)

    pltpu.sync_copy(ctx_hbm.at[pl.ds(base_w, _C)], idxb.at[0])
    pltpu.sync_copy(seg_hbm.at[pl.ds(base_w, _C)], segb.at[0])
    _fire_gather(0, 0)
    _fire_idx(1, 1)

    def _pair_body(p, _):
        _iter(2 * p, 0)
        _iter(2 * p + 1, 1)
        return 0
    lax.fori_loop(0, _G // 2, _pair_body, 0)
    _wait_out()


def kernel(context, segs, personas_no_tag, tags, table):
    pe = jnp.asarray(_PE)
    pnt = personas_no_tag.astype(jnp.int32)
    tg = tags.astype(jnp.int32)
    z2 = jnp.zeros((2,), jnp.int32)
    pidx = jnp.concatenate([pnt[0], tg[0], z2, pnt[1], tg[1], z2])
    out = _sc_kernel(
        context.reshape(-1).astype(jnp.int32),
        segs.reshape(-1).astype(jnp.int32),
        pidx,
        table.astype(jnp.float32),
        pe,
    )
    return out.reshape(_B, _L, _D)


# speedup vs baseline: 3.8810x; 1.4594x over previous
import functools
import math

import jax
import jax.numpy as jnp
import numpy as np
from jax import lax
from jax.experimental import pallas as pl
from jax.experimental.pallas import tpu as pltpu
from jax.experimental.pallas import tpu_sc as plsc

_VOCAB = 100000
_D = 64
_B = 4096
_L = 200
_SPE1 = 3
_SPE2 = 4
_N = _B * _L
_NC = 2
_NS = 16
_NW = _NC * _NS
_PER_W = _N // _NW
_C = 256
_G = _PER_W // _C
_SCALE = math.sqrt(_D)


def _positional_encoding_np():
    pos = np.arange(_L, dtype=np.float32)[:, None]
    div = np.exp(np.arange(0, _D, 2, dtype=np.float32) * (-math.log(10000.0) / _D))
    pe = np.zeros((_L, _D), dtype=np.float32)
    pe[:, 0::2] = np.sin(pos * div)
    pe[:, 1::2] = np.cos(pos * div)
    return pe


_PE = _positional_encoding_np()

_mesh = plsc.VectorSubcoreMesh(core_axis_name="c", subcore_axis_name="s")


@functools.partial(
    pl.kernel,
    out_type=jax.ShapeDtypeStruct((_N, _D), jnp.float32),
    mesh=_mesh,
    compiler_params=pltpu.CompilerParams(use_tc_tiling_on_sc=False),
    scratch_types=[
        pltpu.VMEM_SHARED((3 * _L, _D), jnp.float32),
        pltpu.VMEM((2, _C, _D), jnp.float32),
        pltpu.VMEM((2, _C, _D), jnp.float32),
        pltpu.VMEM((2, _C, _D), jnp.float32),
        pltpu.VMEM((2, _C), jnp.int32),
        pltpu.VMEM((2, _C), jnp.int32),
        pltpu.VMEM((2, _C), jnp.int32),
        pltpu.VMEM((72, _D), jnp.float32),
        pltpu.VMEM((2, _D), jnp.float32),
        pltpu.SemaphoreType.DMA,
        pltpu.SemaphoreType.DMA,
        pltpu.SemaphoreType.DMA,
        pltpu.SemaphoreType.DMA,
    ],
)
def _sc_kernel(ctx_hbm, seg_hbm, pidx_hbm, table_hbm, pe_hbm, out_hbm,
               comb_sp, dest, cbuf, obuf, idxb, segb, kbuf, prow, ps,
               isem, gsem, csem, osem):
    cid = lax.axis_index("c")
    sid = lax.axis_index("s")
    wid = sid * _NC + cid

    @pl.when(sid == 0)
    def _build_comb():
        pe_st = dest.at[0, pl.ds(0, _L)]
        t_st = dest.at[1, pl.ds(0, _L)]
        pltpu.sync_copy(pe_hbm, pe_st)
        pltpu.sync_copy(pe_st, comb_sp.at[pl.ds(0, _L)])
        for p in range(2):
            pltpu.sync_copy(pidx_hbm.at[pl.ds(p * 72, 72)],
                            idxb.at[0, pl.ds(0, 72)])
            pltpu.async_copy(table_hbm.at[idxb.at[0, pl.ds(0, 72)]],
                             prow.at[pl.ds(0, 72)], gsem).wait()
            for j in range(4):
                def _acc_body(i, a, j=j):
                    return a + prow[i, pl.ds(j * 16, 16)]
                acc = lax.fori_loop(0, 70, _acc_body,
                                    jnp.zeros((16,), jnp.float32))
                ps[p, pl.ds(j * 16, 16)] = acc
        for p in range(2):
            def _sec_body(l, _, p=p):
                for j in range(4):
                    t_st[l, pl.ds(j * 16, 16)] = (
                        pe_st[l, pl.ds(j * 16, 16)]
                        + ps[p, pl.ds(j * 16, 16)])
                return 0
            lax.fori_loop(0, _L, _sec_body, 0)
            pltpu.sync_copy(t_st, comb_sp.at[pl.ds((1 + p) * _L, _L)])
    plsc.subcore_barrier()

    base_w = wid * _PER_W

    def _fire_idx(q, slot):
        base = base_w + q * _C
        pltpu.async_copy(ctx_hbm.at[pl.ds(base, _C)], idxb.at[slot], isem)
        pltpu.async_copy(seg_hbm.at[pl.ds(base, _C)], segb.at[slot], isem)

    def _wait_idx(slot):
        pltpu.make_async_copy(ctx_hbm.at[pl.ds(0, _C)], idxb.at[slot],
                              isem).wait()
        pltpu.make_async_copy(seg_hbm.at[pl.ds(0, _C)], segb.at[slot],
                              isem).wait()

    def _kpass(q, slot):
        base = base_w + q * _C
        sb = segb.at[slot]
        kb = kbuf.at[slot]

        def _kgrp(t, _):
            r0 = t * 16
            sv = sb[pl.ds(r0, 16)]
            lv = lax.rem(base + r0 + lax.iota(jnp.int32, 16), _L)
            kb[pl.ds(r0, 16)] = lv + jnp.where(
                sv == _SPE1, _L, jnp.where(sv == _SPE2, 2 * _L, 0))
            return 0
        lax.fori_loop(0, _C // 16, _kgrp, 0)

    def _fire_gathers(q, slot):
        for j in range(_C // 128):
            pltpu.async_copy(
                table_hbm.at[idxb.at[slot, pl.ds(j * 128, 128)]],
                dest.at[slot, pl.ds(j * 128, 128)], gsem)
            pltpu.async_copy(
                comb_sp.at[kbuf.at[slot, pl.ds(j * 128, 128)]],
                cbuf.at[slot, pl.ds(j * 128, 128)], csem)

    def _wait_gathers(slot):
        pltpu.make_async_copy(out_hbm.at[pl.ds(0, _C)], dest.at[slot],
                              gsem).wait()
        pltpu.make_async_copy(out_hbm.at[pl.ds(0, _C)], cbuf.at[slot],
                              csem).wait()

    def _fire_out(q, slot):
        base = base_w + q * _C
        pltpu.async_copy(obuf.at[slot], out_hbm.at[pl.ds(base, _C)], osem)

    def _wait_out():
        pltpu.make_async_copy(obuf.at[0], out_hbm.at[pl.ds(0, _C)],
                              osem).wait()

    def _compute(slot):
        db = dest.at[slot]
        cb = cbuf.at[slot]
        ob = obuf.at[slot]

        def _cgrp(t, _):
            r0 = t * 8
            for i in range(8):
                for j in range(4):
                    ob[r0 + i, pl.ds(j * 16, 16)] = (
                        db[r0 + i, pl.ds(j * 16, 16)] * _SCALE
                        + cb[r0 + i, pl.ds(j * 16, 16)])
            return 0
        lax.fori_loop(0, _C // 8, _cgrp, 0)

    def _iter(q, b):
        nb = 1 - b

        @pl.when(q < _G - 1)
        def _():
            _wait_idx(nb)
            _kpass(q + 1, nb)
            _fire_gathers(q + 1, nb)

        _wait_gathers(b)

        @pl.when(q >= 2)
        def _():
            _wait_out()

        _compute(b)

        @pl.when(q < _G - 2)
        def _():
            _fire_idx(q + 2, b)

        _fire_out(q, b)

    pltpu.sync_copy(ctx_hbm.at[pl.ds(base_w, _C)], idxb.at[0])
    pltpu.sync_copy(seg_hbm.at[pl.ds(base_w, _C)], segb.at[0])
    _kpass(0, 0)
    _fire_gathers(0, 0)
    _fire_idx(1, 1)

    def _pair_body(p, _):
        _iter(2 * p, 0)
        _iter(2 * p + 1, 1)
        return 0
    lax.fori_loop(0, _G // 2, _pair_body, 0)
    _wait_out()
    _wait_out()


def kernel(context, segs, personas_no_tag, tags, table):
    pe = jnp.asarray(_PE)
    pnt = personas_no_tag.astype(jnp.int32)
    tg = tags.astype(jnp.int32)
    z2 = jnp.zeros((2,), jnp.int32)
    pidx = jnp.concatenate([pnt[0], tg[0], z2, pnt[1], tg[1], z2])
    out = _sc_kernel(
        context.reshape(-1).astype(jnp.int32),
        segs.reshape(-1).astype(jnp.int32),
        pidx,
        table.astype(jnp.float32),
        pe,
    )
    return out.reshape(_B, _L, _D)


# speedup vs baseline: 3.8837x; 1.0007x over previous
import functools
import math

import jax
import jax.numpy as jnp
import numpy as np
from jax import lax
from jax.experimental import pallas as pl
from jax.experimental.pallas import tpu as pltpu
from jax.experimental.pallas import tpu_sc as plsc

_VOCAB = 100000
_D = 64
_B = 4096
_L = 200
_SPE1 = 3
_SPE2 = 4
_N = _B * _L
_NC = 2
_NS = 16
_NW = _NC * _NS
_PER_W = _N // _NW
_C = 256
_G = _PER_W // _C
_SCALE = math.sqrt(_D)


def _positional_encoding_np():
    pos = np.arange(_L, dtype=np.float32)[:, None]
    div = np.exp(np.arange(0, _D, 2, dtype=np.float32) * (-math.log(10000.0) / _D))
    pe = np.zeros((_L, _D), dtype=np.float32)
    pe[:, 0::2] = np.sin(pos * div)
    pe[:, 1::2] = np.cos(pos * div)
    return pe


_PE = _positional_encoding_np()

_mesh = plsc.VectorSubcoreMesh(core_axis_name="c", subcore_axis_name="s")


@functools.partial(
    pl.kernel,
    out_type=jax.ShapeDtypeStruct((_N // 2, 2 * _D), jnp.float32),
    mesh=_mesh,
    compiler_params=pltpu.CompilerParams(use_tc_tiling_on_sc=False),
    scratch_types=[
        pltpu.VMEM_SHARED((3 * _L, _D), jnp.float32),
        pltpu.VMEM((2, _C, _D), jnp.float32),
        pltpu.VMEM((2, _C, _D), jnp.float32),
        pltpu.VMEM((2, _C // 2, 2 * _D), jnp.float32),
        pltpu.VMEM((2, _C), jnp.int32),
        pltpu.VMEM((2, _C), jnp.int32),
        pltpu.VMEM((2, _C), jnp.int32),
        pltpu.VMEM((72, _D), jnp.float32),
        pltpu.VMEM((2, _D), jnp.float32),
        pltpu.SemaphoreType.DMA,
        pltpu.SemaphoreType.DMA,
        pltpu.SemaphoreType.DMA,
        pltpu.SemaphoreType.DMA,
    ],
)
def _sc_kernel(ctx_hbm, seg_hbm, pidx_hbm, table_hbm, pe_hbm, out_hbm,
               comb_sp, dest, cbuf, obuf, idxb, segb, kbuf, prow, ps,
               isem, gsem, csem, osem):
    cid = lax.axis_index("c")
    sid = lax.axis_index("s")
    wid = sid * _NC + cid

    @pl.when(sid == 0)
    def _build_comb():
        pe_st = dest.at[0, pl.ds(0, _L)]
        t_st = dest.at[1, pl.ds(0, _L)]
        pltpu.sync_copy(pe_hbm, pe_st)
        pltpu.sync_copy(pe_st, comb_sp.at[pl.ds(0, _L)])
        for p in range(2):
            pltpu.sync_copy(pidx_hbm.at[pl.ds(p * 72, 72)],
                            idxb.at[0, pl.ds(0, 72)])
            pltpu.async_copy(table_hbm.at[idxb.at[0, pl.ds(0, 72)]],
                             prow.at[pl.ds(0, 72)], gsem).wait()
            for j in range(4):
                def _acc_body(i, a, j=j):
                    return a + prow[i, pl.ds(j * 16, 16)]
                acc = lax.fori_loop(0, 70, _acc_body,
                                    jnp.zeros((16,), jnp.float32))
                ps[p, pl.ds(j * 16, 16)] = acc
        for p in range(2):
            def _sec_body(l, _, p=p):
                for j in range(4):
                    t_st[l, pl.ds(j * 16, 16)] = (
                        pe_st[l, pl.ds(j * 16, 16)]
                        + ps[p, pl.ds(j * 16, 16)])
                return 0
            lax.fori_loop(0, _L, _sec_body, 0)
            pltpu.sync_copy(t_st, comb_sp.at[pl.ds((1 + p) * _L, _L)])
    plsc.subcore_barrier()

    base_w = wid * _PER_W

    def _fire_idx(q, slot):
        base = base_w + q * _C
        pltpu.async_copy(ctx_hbm.at[pl.ds(base, _C)], idxb.at[slot], isem)
        pltpu.async_copy(seg_hbm.at[pl.ds(base, _C)], segb.at[slot], isem)

    def _wait_idx(slot):
        pltpu.make_async_copy(ctx_hbm.at[pl.ds(0, _C)], idxb.at[slot],
                              isem).wait()
        pltpu.make_async_copy(seg_hbm.at[pl.ds(0, _C)], segb.at[slot],
                              isem).wait()

    def _kpass(q, slot):
        base = base_w + q * _C
        sb = segb.at[slot]
        kb = kbuf.at[slot]

        def _kgrp(t, _):
            r0 = t * 16
            sv = sb[pl.ds(r0, 16)]
            lv = lax.rem(base + r0 + lax.iota(jnp.int32, 16), _L)
            kb[pl.ds(r0, 16)] = lv + jnp.where(
                sv == _SPE1, _L, jnp.where(sv == _SPE2, 2 * _L, 0))
            return 0
        lax.fori_loop(0, _C // 16, _kgrp, 0)

    def _fire_gathers(q, slot):
        for j in range(_C // 128):
            pltpu.async_copy(
                table_hbm.at[idxb.at[slot, pl.ds(j * 128, 128)]],
                dest.at[slot, pl.ds(j * 128, 128)], gsem)
            pltpu.async_copy(
                comb_sp.at[kbuf.at[slot, pl.ds(j * 128, 128)]],
                cbuf.at[slot, pl.ds(j * 128, 128)], csem)

    def _wait_gathers(slot):
        pltpu.make_async_copy(table_hbm.at[pl.ds(0, _C)], dest.at[slot],
                              gsem).wait()
        pltpu.make_async_copy(table_hbm.at[pl.ds(0, _C)], cbuf.at[slot],
                              csem).wait()

    def _fire_out(q, slot):
        base2 = (base_w + q * _C) // 2
        pltpu.async_copy(obuf.at[slot], out_hbm.at[pl.ds(base2, _C // 2)],
                         osem)

    def _wait_out():
        pltpu.make_async_copy(obuf.at[0], out_hbm.at[pl.ds(0, _C // 2)],
                              osem).wait()

    def _compute(slot):
        db = dest.at[slot]
        cb = cbuf.at[slot]
        ob = obuf.at[slot]

        def _cgrp(t, _):
            r0 = t * 8
            r2 = t * 4
            for i in range(8):
                for j in range(4):
                    ob[r2 + i // 2, pl.ds((i % 2) * _D + j * 16, 16)] = (
                        db[r0 + i, pl.ds(j * 16, 16)] * _SCALE
                        + cb[r0 + i, pl.ds(j * 16, 16)])
            return 0
        lax.fori_loop(0, _C // 8, _cgrp, 0)

    def _iter(q, b):
        nb = 1 - b

        @pl.when(q < _G - 1)
        def _():
            _wait_idx(nb)
            _kpass(q + 1, nb)
            _fire_gathers(q + 1, nb)

        _wait_gathers(b)

        @pl.when(q >= 2)
        def _():
            _wait_out()

        _compute(b)

        @pl.when(q < _G - 2)
        def _():
            _fire_idx(q + 2, b)

        _fire_out(q, b)

    pltpu.sync_copy(ctx_hbm.at[pl.ds(base_w, _C)], idxb.at[0])
    pltpu.sync_copy(seg_hbm.at[pl.ds(base_w, _C)], segb.at[0])
    _kpass(0, 0)
    _fire_gathers(0, 0)
    _fire_idx(1, 1)

    def _pair_body(p, _):
        _iter(2 * p, 0)
        _iter(2 * p + 1, 1)
        return 0
    lax.fori_loop(0, _G // 2, _pair_body, 0)
    _wait_out()
    _wait_out()


def kernel(context, segs, personas_no_tag, tags, table):
    pe = jnp.asarray(_PE)
    pnt = personas_no_tag.astype(jnp.int32)
    tg = tags.astype(jnp.int32)
    z2 = jnp.zeros((2,), jnp.int32)
    pidx = jnp.concatenate([pnt[0], tg[0], z2, pnt[1], tg[1], z2])
    out = _sc_kernel(
        context.reshape(-1).astype(jnp.int32),
        segs.reshape(-1).astype(jnp.int32),
        pidx,
        table.astype(jnp.float32),
        pe,
    )
    return out.reshape(_B, _L, _D)
